# baseline (speedup 1.0000x reference)
import functools

import jax
import jax.numpy as jnp
from jax import lax
from jax.experimental import pallas as pl
from jax.experimental.pallas import tpu as pltpu
from jax.experimental.pallas import tpu_sc as plsc

NC = 2
NS = 16
NW = NC * NS
C = 128


def _matmul_body(x_ref, w_ref, o_ref):
    o_ref[...] = jnp.dot(x_ref[...], w_ref[...],
                         preferred_element_type=jnp.float32)


def _gru_body(x_ref, p0_ref, p1_ref, wih_ref, whh_ref, bi_ref, bh_ref, o_ref):
    d = x_ref.shape[1]
    xb = x_ref[...]
    agg = p0_ref[...] + p1_ref[...]
    gi = jnp.dot(agg, wih_ref[...], preferred_element_type=jnp.float32) + bi_ref[...]
    gh = jnp.dot(xb, whh_ref[...], preferred_element_type=jnp.float32) + bh_ref[...]
    i_r, i_z, i_n = gi[:, :d], gi[:, d:2 * d], gi[:, 2 * d:]
    h_r, h_z, h_n = gh[:, :d], gh[:, d:2 * d], gh[:, 2 * d:]
    r = jax.nn.sigmoid(i_r + h_r)
    z = jax.nn.sigmoid(i_z + h_z)
    n = jnp.tanh(i_n + r * h_n)
    h_new = (1.0 - z) * n + z * xb
    o_ref[...] = xb + jnp.maximum(h_new, 0.0)


NBUF = 2


def _make_scatter_kernel(n_agg, d, n_groups, rows_per_tile):
    mesh = plsc.VectorSubcoreMesh(core_axis_name="c", subcore_axis_name="s",
                                  num_cores=NC, num_subcores=NS)

    @functools.partial(
        pl.kernel,
        out_type=jax.ShapeDtypeStruct((NC, n_agg, d), jnp.float32),
        mesh=mesh,
        scratch_types=[
            pltpu.VMEM_SHARED((n_agg, d), jnp.float32),
            pltpu.VMEM((2, NBUF, 2, C), jnp.int32),
            pltpu.VMEM((NBUF, C, d), jnp.float32),
        ] + [pltpu.SemaphoreType.DMA] * (NBUF + 1),
    )
    def scatter_kernel(m_hbm, e_hbm, zeros_hbm, out_hbm,
                       agg_sp, idx_v, rows_v, *sems):
        c = lax.axis_index("c")
        s = lax.axis_index("s")
        isem = sems[NBUF]
        base = s * rows_per_tile
        pltpu.sync_copy(zeros_hbm.at[pl.ds(base, rows_per_tile)],
                        agg_sp.at[pl.ds(base, rows_per_tile)])
        plsc.subcore_barrier()

        pltpu.sync_copy(e_hbm.at[c, s, 0], idx_v.at[0])
        for b in range(NBUF):
            pltpu.async_copy(m_hbm.at[idx_v.at[0, b, 0]], rows_v.at[b],
                             sems[b])
        pltpu.async_copy(e_hbm.at[c, s, 1], idx_v.at[1], isem)

        def process_group(g, ring, nring, regather, prefetch):
            for b in range(NBUF):
                pltpu.make_async_copy(m_hbm.at[idx_v.at[ring, b, 0]],
                                      rows_v.at[b], sems[b]).wait()
                pltpu.sync_copy(rows_v.at[b], agg_sp.at[idx_v.at[ring, b, 1]],
                                add=True)
                if regather:
                    pltpu.async_copy(m_hbm.at[idx_v.at[nring, b, 0]],
                                     rows_v.at[b], sems[b])
            if prefetch:
                pltpu.async_copy(e_hbm.at[c, s, g + 2], idx_v.at[ring], isem)

        def outer(g, carry):
            ring = lax.rem(g, 2)
            nring = 1 - ring
            pltpu.make_async_copy(e_hbm.at[c, s, g + 1], idx_v.at[nring],
                                  isem).wait()
            process_group(g, ring, nring, regather=True, prefetch=True)
            return carry

        lax.fori_loop(0, n_groups - 2, outer, 0)
        g = n_groups - 2
        ring = g % 2
        pltpu.make_async_copy(e_hbm.at[c, s, g + 1], idx_v.at[1 - ring],
                              isem).wait()
        process_group(g, ring, 1 - ring, regather=True, prefetch=False)
        process_group(g + 1, 1 - ring, ring, regather=False, prefetch=False)

        plsc.subcore_barrier()
        pltpu.sync_copy(agg_sp.at[pl.ds(base, rows_per_tile)],
                        out_hbm.at[c, pl.ds(base, rows_per_tile)])

    return scatter_kernel


def kernel(x, edge_index, weight, w_ih, w_hh, b_ih, b_hh):
    n, d = x.shape
    e = edge_index.shape[1]

    per_w = -(-e // NW)
    chunks = -(-(-(-per_w // C)) // NBUF) * NBUF
    n_groups = chunks // NBUF
    e_pad = NW * chunks * C
    dummy_dst = n
    n_agg = -(-(n + 1) // (NS * 8)) * (NS * 8)
    rows_per_tile = n_agg // NS

    src = jnp.concatenate(
        [edge_index[0], jnp.zeros((e_pad - e,), jnp.int32)]).reshape(
            NC, NS, n_groups, NBUF, 1, C)
    dst = jnp.concatenate(
        [edge_index[1], jnp.full((e_pad - e,), dummy_dst, jnp.int32)]).reshape(
            NC, NS, n_groups, NBUF, 1, C)
    e_pack = jnp.concatenate([src, dst], axis=4)
    zeros_hbm = jnp.zeros((n_agg, d), jnp.float32)

    br = 2000
    m = pl.pallas_call(
        _matmul_body,
        grid=(n // br,),
        in_specs=[pl.BlockSpec((br, d), lambda i: (i, 0)),
                  pl.BlockSpec((d, d), lambda i: (0, 0))],
        out_specs=pl.BlockSpec((br, d), lambda i: (i, 0)),
        out_shape=jax.ShapeDtypeStruct((n, d), jnp.float32),
    )(x, weight[0])

    partial = _make_scatter_kernel(n_agg, d, n_groups, rows_per_tile)(
        m, e_pack, zeros_hbm)

    out = pl.pallas_call(
        _gru_body,
        grid=(n // br,),
        in_specs=[
            pl.BlockSpec((br, d), lambda i: (i, 0)),
            pl.BlockSpec((br, d), lambda i: (i, 0)),
            pl.BlockSpec((br, d), lambda i: (i, 0)),
            pl.BlockSpec((d, 3 * d), lambda i: (0, 0)),
            pl.BlockSpec((d, 3 * d), lambda i: (0, 0)),
            pl.BlockSpec((1, 3 * d), lambda i: (0, 0)),
            pl.BlockSpec((1, 3 * d), lambda i: (0, 0)),
        ],
        out_specs=pl.BlockSpec((br, d), lambda i: (i, 0)),
        out_shape=jax.ShapeDtypeStruct((n, d), jnp.float32),
    )(x, partial[0, :n], partial[1, :n], w_ih.T, w_hh.T,
      b_ih.reshape(1, -1), b_hh.reshape(1, -1))

    return out


# speedup vs baseline: 1.7249x; 1.7249x over previous
import functools

import jax
import jax.numpy as jnp
from jax import lax
from jax.experimental import pallas as pl
from jax.experimental.pallas import tpu as pltpu
from jax.experimental.pallas import tpu_sc as plsc

NC = 2
NS = 16
C = 128
NBUF = 2
SPLIT0 = 0.80


def _pre_body(x_ref, w_ref, bh_ref, m_ref, gh_ref):
    d = x_ref.shape[1]
    prod = jnp.dot(x_ref[...], w_ref[...], preferred_element_type=jnp.float32)
    m_ref[...] = prod[:, :d]
    gh_ref[...] = prod[:, d:] + bh_ref[...]


def _gru_body(x_ref, p_ref, wih_ref, bi_ref, gh_ref, o_ref):
    d = x_ref.shape[1]
    xb = x_ref[...]
    agg = p_ref[0] + p_ref[1]
    gi = jnp.dot(agg, wih_ref[...], preferred_element_type=jnp.float32) + bi_ref[...]
    gh = gh_ref[...]
    i_r, i_z, i_n = gi[:, :d], gi[:, d:2 * d], gi[:, 2 * d:]
    h_r, h_z, h_n = gh[:, :d], gh[:, d:2 * d], gh[:, 2 * d:]
    r = jax.nn.sigmoid(i_r + h_r)
    z = jax.nn.sigmoid(i_z + h_z)
    n = jnp.tanh(i_n + r * h_n)
    h_new = (1.0 - z) * n + z * xb
    o_ref[...] = xb + jnp.maximum(h_new, 0.0)


def _make_scatter_kernel(n_agg, d, g0, g1, rows_per_tile):
    mesh = plsc.VectorSubcoreMesh(core_axis_name="c", subcore_axis_name="s",
                                  num_cores=NC, num_subcores=NS)

    @functools.partial(
        pl.kernel,
        out_type=jax.ShapeDtypeStruct((NC, n_agg, d), jnp.float32),
        mesh=mesh,
        scratch_types=[
            pltpu.VMEM_SHARED((n_agg, d), jnp.float32),
            pltpu.VMEM((2, NBUF, 2, C), jnp.int32),
            pltpu.VMEM((NBUF, C, d), jnp.float32),
        ] + [pltpu.SemaphoreType.DMA] * (NBUF + 1),
    )
    def scatter_kernel(m_hbm, e_hbm, zeros_hbm, out_hbm,
                       agg_sp, idx_v, rows_v, *sems):
        c = lax.axis_index("c")
        s = lax.axis_index("s")
        isem = sems[NBUF]
        base = s * rows_per_tile
        n_groups = lax.select(c == 0, jnp.int32(g0), jnp.int32(g1))
        pltpu.sync_copy(zeros_hbm.at[pl.ds(base, rows_per_tile)],
                        agg_sp.at[pl.ds(base, rows_per_tile)])
        plsc.subcore_barrier()

        pltpu.sync_copy(e_hbm.at[c, s, 0], idx_v.at[0])
        for b in range(NBUF):
            pltpu.async_copy(m_hbm.at[idx_v.at[0, b, 0]], rows_v.at[b],
                             sems[b])
        pltpu.async_copy(e_hbm.at[c, s, 1], idx_v.at[1], isem)

        def process_group(g, ring, nring, regather, prefetch):
            for b in range(NBUF):
                pltpu.make_async_copy(m_hbm.at[idx_v.at[ring, b, 0]],
                                      rows_v.at[b], sems[b]).wait()
                pltpu.sync_copy(rows_v.at[b], agg_sp.at[idx_v.at[ring, b, 1]],
                                add=True)
                if regather:
                    pltpu.async_copy(m_hbm.at[idx_v.at[nring, b, 0]],
                                     rows_v.at[b], sems[b])
            if prefetch:
                pltpu.async_copy(e_hbm.at[c, s, g + 2], idx_v.at[ring], isem)

        def outer(g, carry):
            ring = lax.rem(g, 2)
            nring = 1 - ring
            pltpu.make_async_copy(e_hbm.at[c, s, g + 1], idx_v.at[nring],
                                  isem).wait()
            process_group(g, ring, nring, regather=True, prefetch=True)
            return carry

        lax.fori_loop(0, n_groups - 2, outer, jnp.int32(0))
        g = n_groups - 2
        ring = lax.rem(g, 2)
        nring = 1 - ring
        pltpu.make_async_copy(e_hbm.at[c, s, g + 1], idx_v.at[nring],
                              isem).wait()
        process_group(g, ring, nring, regather=True, prefetch=False)
        process_group(g + 1, nring, ring, regather=False, prefetch=False)

        plsc.subcore_barrier()
        pltpu.sync_copy(agg_sp.at[pl.ds(base, rows_per_tile)],
                        out_hbm.at[c, pl.ds(base, rows_per_tile)])

    return scatter_kernel


def kernel(x, edge_index, weight, w_ih, w_hh, b_ih, b_hh):
    n, d = x.shape
    e = edge_index.shape[1]

    per_group = NS * NBUF * C
    g_tot = -(-e // per_group)
    g0 = max(2, min(g_tot - 2, round(g_tot * SPLIT0)))
    g1 = g_tot - g0
    e0 = g0 * per_group
    e_pad = g_tot * per_group
    dummy_dst = n
    n_agg = -(-(n + 1) // (NS * 8)) * (NS * 8)
    rows_per_tile = n_agg // NS

    src = jnp.concatenate(
        [edge_index[0], jnp.zeros((e_pad - e,), jnp.int32)])
    dst = jnp.concatenate(
        [edge_index[1], jnp.full((e_pad - e,), dummy_dst, jnp.int32)])

    def pack(sr, ds, g):
        sr = sr.reshape(NS, g, NBUF, 1, C)
        ds = ds.reshape(NS, g, NBUF, 1, C)
        return jnp.concatenate([sr, ds], axis=3)

    pack0 = pack(src[:e0], dst[:e0], g0)
    pack1 = pack(src[e0:], dst[e0:], g1)
    pack1 = jnp.pad(pack1, ((0, 0), (0, g0 - g1), (0, 0), (0, 0), (0, 0)))
    e_pack = jnp.stack([pack0, pack1])
    zeros_hbm = jnp.zeros((n_agg, d), jnp.float32)

    br = 2000
    wcat = jnp.concatenate([weight[0], w_hh.T], axis=1)
    m, gh = pl.pallas_call(
        _pre_body,
        grid=(n // br,),
        in_specs=[pl.BlockSpec((br, d), lambda i: (i, 0)),
                  pl.BlockSpec((d, 4 * d), lambda i: (0, 0)),
                  pl.BlockSpec((1, 3 * d), lambda i: (0, 0))],
        out_specs=[pl.BlockSpec((br, d), lambda i: (i, 0)),
                   pl.BlockSpec((br, 3 * d), lambda i: (i, 0))],
        out_shape=[jax.ShapeDtypeStruct((n, d), jnp.float32),
                   jax.ShapeDtypeStruct((n, 3 * d), jnp.float32)],
    )(x, wcat, b_hh.reshape(1, -1))

    partial = _make_scatter_kernel(n_agg, d, g0, g1, rows_per_tile)(
        m, e_pack, zeros_hbm)

    out = pl.pallas_call(
        _gru_body,
        grid=(n // br,),
        in_specs=[
            pl.BlockSpec((br, d), lambda i: (i, 0)),
            pl.BlockSpec((2, br, d), lambda i: (0, i, 0)),
            pl.BlockSpec((d, 3 * d), lambda i: (0, 0)),
            pl.BlockSpec((1, 3 * d), lambda i: (0, 0)),
            pl.BlockSpec((br, 3 * d), lambda i: (i, 0)),
        ],
        out_specs=pl.BlockSpec((br, d), lambda i: (i, 0)),
        out_shape=jax.ShapeDtypeStruct((n, d), jnp.float32),
    )(x, partial, w_ih.T, b_ih.reshape(1, -1), gh)

    return out


# speedup vs baseline: 1.8721x; 1.0854x over previous
import functools

import jax
import jax.numpy as jnp
from jax import lax
from jax.experimental import pallas as pl
from jax.experimental.pallas import tpu as pltpu
from jax.experimental.pallas import tpu_sc as plsc

NC = 2
NS = 16
C = 128
NBUF = 2
SPLIT0 = 0.829


def _matmul_body(x_ref, w_ref, o_ref):
    o_ref[...] = jnp.dot(x_ref[...], w_ref[...],
                         preferred_element_type=jnp.float32)


def _gru_body(x_ref, p_ref, wih_ref, whh_ref, bi_ref, bh_ref, o_ref):
    d = x_ref.shape[1]
    xb = x_ref[...]
    agg = p_ref[0] + p_ref[1]
    gi = jnp.dot(agg, wih_ref[...], preferred_element_type=jnp.float32) + bi_ref[...]
    gh = jnp.dot(xb, whh_ref[...], preferred_element_type=jnp.float32) + bh_ref[...]
    i_r, i_z, i_n = gi[:, :d], gi[:, d:2 * d], gi[:, 2 * d:]
    h_r, h_z, h_n = gh[:, :d], gh[:, d:2 * d], gh[:, 2 * d:]
    r = jax.nn.sigmoid(i_r + h_r)
    z = jax.nn.sigmoid(i_z + h_z)
    n = jnp.tanh(i_n + r * h_n)
    h_new = (1.0 - z) * n + z * xb
    o_ref[...] = xb + jnp.maximum(h_new, 0.0)


def _make_scatter_kernel(n_agg, d, g0, g1, rows_per_tile):
    mesh = plsc.VectorSubcoreMesh(core_axis_name="c", subcore_axis_name="s",
                                  num_cores=NC, num_subcores=NS)

    @functools.partial(
        pl.kernel,
        out_type=jax.ShapeDtypeStruct((NC, n_agg, d), jnp.float32),
        mesh=mesh,
        scratch_types=[
            pltpu.VMEM_SHARED((n_agg, d), jnp.float32),
            pltpu.VMEM((2, 2, NBUF, C), jnp.int32),
            pltpu.VMEM((NBUF, C, d), jnp.float32),
        ] + [pltpu.SemaphoreType.DMA] * (NBUF + 1),
    )
    def scatter_kernel(m_hbm, e_hbm, zeros_hbm, out_hbm,
                       agg_sp, idx_v, rows_v, *sems):
        c = lax.axis_index("c")
        s = lax.axis_index("s")
        isem = sems[NBUF]
        base = s * rows_per_tile
        n_groups = lax.select(c == 0, jnp.int32(g0), jnp.int32(g1))
        k0 = lax.select(c == 0, s * (g0 * NBUF),
                        NS * (g0 * NBUF) + s * (g1 * NBUF))

        def load_idx(g, slot):
            for io in range(2):
                for b in range(NBUF):
                    pltpu.async_copy(e_hbm.at[io, k0 + g * NBUF + b],
                                     idx_v.at[slot, io, b], isem)

        def wait_idx():
            for io in range(2):
                for b in range(NBUF):
                    pltpu.make_async_copy(e_hbm.at[io, 0], idx_v.at[0, io, b],
                                          isem).wait()

        pltpu.sync_copy(zeros_hbm.at[pl.ds(base, rows_per_tile)],
                        agg_sp.at[pl.ds(base, rows_per_tile)])
        plsc.subcore_barrier()

        load_idx(0, 0)
        wait_idx()
        for b in range(NBUF):
            pltpu.async_copy(m_hbm.at[idx_v.at[0, 0, b]], rows_v.at[b],
                             sems[b])
        load_idx(1, 1)

        def process_group(g, ring, nring, regather, prefetch):
            for b in range(NBUF):
                pltpu.make_async_copy(m_hbm.at[idx_v.at[ring, 0, b]],
                                      rows_v.at[b], sems[b]).wait()
                pltpu.sync_copy(rows_v.at[b], agg_sp.at[idx_v.at[ring, 1, b]],
                                add=True)
                if regather:
                    pltpu.async_copy(m_hbm.at[idx_v.at[nring, 0, b]],
                                     rows_v.at[b], sems[b])
            if prefetch:
                load_idx(g + 2, ring)

        def outer(g, carry):
            ring = lax.rem(g, 2)
            nring = 1 - ring
            wait_idx()
            process_group(g, ring, nring, regather=True, prefetch=True)
            return carry

        lax.fori_loop(0, n_groups - 2, outer, jnp.int32(0))
        g = n_groups - 2
        ring = lax.rem(g, 2)
        nring = 1 - ring
        wait_idx()
        process_group(g, ring, nring, regather=True, prefetch=False)
        process_group(g + 1, nring, ring, regather=False, prefetch=False)

        plsc.subcore_barrier()
        pltpu.sync_copy(agg_sp.at[pl.ds(base, rows_per_tile)],
                        out_hbm.at[c, pl.ds(base, rows_per_tile)])

    return scatter_kernel


def kernel(x, edge_index, weight, w_ih, w_hh, b_ih, b_hh):
    n, d = x.shape
    e = edge_index.shape[1]

    per_group = NS * NBUF * C
    g_tot = -(-e // per_group)
    g0 = max(2, min(g_tot - 2, round(g_tot * SPLIT0)))
    g1 = g_tot - g0
    e_pad = g_tot * per_group
    dummy_dst = n
    n_agg = -(-(n + 1) // (NS * 8)) * (NS * 8)
    rows_per_tile = n_agg // NS

    pad = jnp.zeros((2, e_pad - e), jnp.int32).at[1, :].set(dummy_dst)
    e_hbm = jnp.concatenate([edge_index, pad], axis=1).reshape(
        2, e_pad // C, C)
    zeros_hbm = jnp.zeros((n_agg, d), jnp.float32)

    br = 2000
    m = pl.pallas_call(
        _matmul_body,
        grid=(n // br,),
        in_specs=[pl.BlockSpec((br, d), lambda i: (i, 0)),
                  pl.BlockSpec((d, d), lambda i: (0, 0))],
        out_specs=pl.BlockSpec((br, d), lambda i: (i, 0)),
        out_shape=jax.ShapeDtypeStruct((n, d), jnp.float32),
    )(x, weight[0])

    partial = _make_scatter_kernel(n_agg, d, g0, g1, rows_per_tile)(
        m, e_hbm, zeros_hbm)

    out = pl.pallas_call(
        _gru_body,
        grid=(n // br,),
        in_specs=[
            pl.BlockSpec((br, d), lambda i: (i, 0)),
            pl.BlockSpec((2, br, d), lambda i: (0, i, 0)),
            pl.BlockSpec((d, 3 * d), lambda i: (0, 0)),
            pl.BlockSpec((d, 3 * d), lambda i: (0, 0)),
            pl.BlockSpec((1, 3 * d), lambda i: (0, 0)),
            pl.BlockSpec((1, 3 * d), lambda i: (0, 0)),
        ],
        out_specs=pl.BlockSpec((br, d), lambda i: (i, 0)),
        out_shape=jax.ShapeDtypeStruct((n, d), jnp.float32),
    )(x, partial, w_ih.T, w_hh.T, b_ih.reshape(1, -1), b_hh.reshape(1, -1))

    return out


# speedup vs baseline: 2.2283x; 1.1903x over previous
import functools

import jax
import jax.numpy as jnp
from jax import lax
from jax.experimental import pallas as pl
from jax.experimental.pallas import tpu as pltpu
from jax.experimental.pallas import tpu_sc as plsc

NC = 2
NS = 16
C = 80
NBUF = 4
NSLOT = 3
SPLIT0 = 0.829


def _matmul_body(x_ref, w_ref, o_ref):
    o_ref[...] = jnp.dot(x_ref[...], w_ref[...],
                         preferred_element_type=jnp.float32)


def _gru_body(x_ref, p_ref, wih_ref, whh_ref, bi_ref, bh_ref, o_ref):
    d = x_ref.shape[1]
    xb = x_ref[...]
    agg = p_ref[0] + p_ref[1]
    gi = jnp.dot(agg, wih_ref[...], preferred_element_type=jnp.float32) + bi_ref[...]
    gh = jnp.dot(xb, whh_ref[...], preferred_element_type=jnp.float32) + bh_ref[...]
    i_r, i_z, i_n = gi[:, :d], gi[:, d:2 * d], gi[:, 2 * d:]
    h_r, h_z, h_n = gh[:, :d], gh[:, d:2 * d], gh[:, 2 * d:]
    r = jax.nn.sigmoid(i_r + h_r)
    z = jax.nn.sigmoid(i_z + h_z)
    n = jnp.tanh(i_n + r * h_n)
    h_new = (1.0 - z) * n + z * xb
    o_ref[...] = xb + jnp.maximum(h_new, 0.0)


def _make_scatter_kernel(n_agg, d, g0, g1, rows_per_tile):
    mesh = plsc.VectorSubcoreMesh(core_axis_name="c", subcore_axis_name="s",
                                  num_cores=NC, num_subcores=NS)

    @functools.partial(
        pl.kernel,
        out_type=jax.ShapeDtypeStruct((NC, n_agg, d), jnp.float32),
        mesh=mesh,
        scratch_types=[
            pltpu.VMEM_SHARED((n_agg, d), jnp.float32),
            pltpu.VMEM((NSLOT, 2, NBUF, C), jnp.int32),
            pltpu.VMEM((NBUF, C, d), jnp.float32),
        ] + [pltpu.SemaphoreType.DMA] * (NBUF + NSLOT),
    )
    def scatter_kernel(m_hbm, e_hbm, zeros_hbm, out_hbm,
                       agg_sp, idx_v, rows_v, *sems):
        c = lax.axis_index("c")
        s = lax.axis_index("s")
        gsems, isems = sems[:NBUF], sems[NBUF:]
        base = s * rows_per_tile
        n_groups = lax.select(c == 0, jnp.int32(g0), jnp.int32(g1))
        k0 = lax.select(c == 0, s * (g0 * NBUF),
                        NS * (g0 * NBUF) + s * (g1 * NBUF))

        def load_idx(g, slot, isem):
            for io in range(2):
                for b in range(NBUF):
                    pltpu.async_copy(e_hbm.at[io, k0 + g * NBUF + b],
                                     idx_v.at[slot, io, b], isem)

        def wait_idx(isem):
            for io in range(2):
                for b in range(NBUF):
                    pltpu.make_async_copy(e_hbm.at[io, 0], idx_v.at[0, io, b],
                                          isem).wait()

        pltpu.sync_copy(zeros_hbm.at[pl.ds(base, rows_per_tile)],
                        agg_sp.at[pl.ds(base, rows_per_tile)])
        plsc.subcore_barrier()

        for g in range(NSLOT):
            load_idx(g, g, isems[g])
        wait_idx(isems[0])
        for b in range(NBUF):
            pltpu.async_copy(m_hbm.at[idx_v.at[0, 0, b]], rows_v.at[b],
                             gsems[b])

        def process_group(ring, nring, regather):
            for b in range(NBUF):
                pltpu.make_async_copy(m_hbm.at[idx_v.at[ring, 0, b]],
                                      rows_v.at[b], gsems[b]).wait()
                pltpu.sync_copy(rows_v.at[b], agg_sp.at[idx_v.at[ring, 1, b]],
                                add=True)
                if regather:
                    pltpu.async_copy(m_hbm.at[idx_v.at[nring, 0, b]],
                                     rows_v.at[b], gsems[b])

        def outer(t, carry):
            for ph in range(NSLOT):
                g = t * NSLOT + ph
                ring, nring = ph, (ph + 1) % NSLOT
                wait_idx(isems[nring])
                process_group(ring, nring, regather=True)
                load_idx(g + NSLOT, ring, isems[ring])
            return carry

        n_full = (n_groups - NSLOT) // NSLOT
        lax.fori_loop(0, n_full, outer, jnp.int32(0))

        g_done = n_full * NSLOT
        n_rem = n_groups - g_done

        for ph in range(2 * NSLOT - 1):
            ring, nring = ph % NSLOT, (ph + 1) % NSLOT

            @pl.when(ph + 1 < n_rem)
            def _(ring=ring, nring=nring):
                wait_idx(isems[nring])
                process_group(ring, nring, regather=True)

            @pl.when(ph + 1 == n_rem)
            def _(ring=ring, nring=nring):
                process_group(ring, nring, regather=False)

            @pl.when(ph + NSLOT < n_rem)
            def _(ph=ph, ring=ring):
                load_idx(g_done + ph + NSLOT, ring, isems[ring])

        plsc.subcore_barrier()
        pltpu.sync_copy(agg_sp.at[pl.ds(base, rows_per_tile)],
                        out_hbm.at[c, pl.ds(base, rows_per_tile)])

    return scatter_kernel


def kernel(x, edge_index, weight, w_ih, w_hh, b_ih, b_hh):
    n, d = x.shape
    e = edge_index.shape[1]

    per_group = NS * NBUF * C
    g_tot = -(-e // per_group)
    g0 = max(NSLOT, min(g_tot - NSLOT, round(g_tot * SPLIT0)))
    g1 = g_tot - g0
    e_pad = g_tot * per_group
    dummy_dst = n
    n_agg = -(-(n + 1) // (NS * 8)) * (NS * 8)
    rows_per_tile = n_agg // NS

    pad = jnp.zeros((2, e_pad - e), jnp.int32).at[1, :].set(dummy_dst)
    e_hbm = jnp.concatenate([edge_index, pad], axis=1).reshape(
        2, e_pad // C, C)
    zeros_hbm = jnp.zeros((n_agg, d), jnp.float32)

    br = 2000
    m = pl.pallas_call(
        _matmul_body,
        grid=(n // br,),
        in_specs=[pl.BlockSpec((br, d), lambda i: (i, 0)),
                  pl.BlockSpec((d, d), lambda i: (0, 0))],
        out_specs=pl.BlockSpec((br, d), lambda i: (i, 0)),
        out_shape=jax.ShapeDtypeStruct((n, d), jnp.float32),
    )(x, weight[0])

    partial = _make_scatter_kernel(n_agg, d, g0, g1, rows_per_tile)(
        m, e_hbm, zeros_hbm)

    out = pl.pallas_call(
        _gru_body,
        grid=(n // br,),
        in_specs=[
            pl.BlockSpec((br, d), lambda i: (i, 0)),
            pl.BlockSpec((2, br, d), lambda i: (0, i, 0)),
            pl.BlockSpec((d, 3 * d), lambda i: (0, 0)),
            pl.BlockSpec((d, 3 * d), lambda i: (0, 0)),
            pl.BlockSpec((1, 3 * d), lambda i: (0, 0)),
            pl.BlockSpec((1, 3 * d), lambda i: (0, 0)),
        ],
        out_specs=pl.BlockSpec((br, d), lambda i: (i, 0)),
        out_shape=jax.ShapeDtypeStruct((n, d), jnp.float32),
    )(x, partial, w_ih.T, w_hh.T, b_ih.reshape(1, -1), b_hh.reshape(1, -1))

    return out


# speedup vs baseline: 2.8396x; 1.2743x over previous
import functools

import jax
import jax.numpy as jnp
from jax import lax
from jax.experimental import pallas as pl
from jax.experimental.pallas import tpu as pltpu
from jax.experimental.pallas import tpu_sc as plsc

NC = 2
NS = 16
C = 80
NBUF = 4
NSLOT = 3
SPLIT0 = 0.829


def _matmul_body(x_ref, w_ref, o_ref, o2_ref):
    prod = jnp.dot(x_ref[...], w_ref[...], preferred_element_type=jnp.float32)
    o_ref[...] = prod
    o2_ref[...] = prod


def _gru_body(x_ref, p_ref, wih_ref, whh_ref, bi_ref, bh_ref, o_ref):
    d = x_ref.shape[1]
    xb = x_ref[...]
    agg = p_ref[0] + p_ref[1]
    gi = jnp.dot(agg, wih_ref[...], preferred_element_type=jnp.float32) + bi_ref[...]
    gh = jnp.dot(xb, whh_ref[...], preferred_element_type=jnp.float32) + bh_ref[...]
    i_r, i_z, i_n = gi[:, :d], gi[:, d:2 * d], gi[:, 2 * d:]
    h_r, h_z, h_n = gh[:, :d], gh[:, d:2 * d], gh[:, 2 * d:]
    r = jax.nn.sigmoid(i_r + h_r)
    z = jax.nn.sigmoid(i_z + h_z)
    n = jnp.tanh(i_n + r * h_n)
    h_new = (1.0 - z) * n + z * xb
    o_ref[...] = xb + jnp.maximum(h_new, 0.0)


def _make_scatter_kernel(n_agg, d, q0, r0, g0_total, q1, r1, rows_per_tile):
    mesh = plsc.VectorSubcoreMesh(core_axis_name="c", subcore_axis_name="s",
                                  num_cores=NC, num_subcores=NS)

    @functools.partial(
        pl.kernel,
        out_type=jax.ShapeDtypeStruct((NC, n_agg, d), jnp.float32),
        mesh=mesh,
        scratch_types=[
            pltpu.VMEM_SHARED((n_agg, d), jnp.float32),
            pltpu.VMEM((NSLOT, 2, NBUF, C), jnp.int32),
            pltpu.VMEM((NBUF, C, d), jnp.float32),
        ] + [pltpu.SemaphoreType.DMA] * (NBUF + NSLOT),
    )
    def scatter_kernel(m0_hbm, m1_hbm, e_hbm, zeros_hbm, out_hbm,
                       agg_sp, idx_v, rows_v, *sems):
        c = lax.axis_index("c")
        s = lax.axis_index("s")
        gsems, isems = sems[:NBUF], sems[NBUF:]
        base = s * rows_per_tile
        n_groups = lax.select(c == 0, q0 + (s < r0).astype(jnp.int32),
                              q1 + (s < r1).astype(jnp.int32))
        start_g = lax.select(
            c == 0, s * q0 + jnp.minimum(s, r0),
            g0_total + s * q1 + jnp.minimum(s, r1))
        k0 = start_g * NBUF

        def load_idx(g, slot, isem):
            for io in range(2):
                for b in range(NBUF):
                    pltpu.async_copy(e_hbm.at[io, k0 + g * NBUF + b],
                                     idx_v.at[slot, io, b], isem)

        def wait_idx(isem):
            for io in range(2):
                for b in range(NBUF):
                    pltpu.make_async_copy(e_hbm.at[io, 0], idx_v.at[0, io, b],
                                          isem).wait()

        def gather(slot, b, buf):
            @pl.when(c == 0)
            def _():
                pltpu.async_copy(m0_hbm.at[idx_v.at[slot, 0, b]],
                                 rows_v.at[buf], gsems[buf])

            @pl.when(c != 0)
            def _():
                pltpu.async_copy(m1_hbm.at[idx_v.at[slot, 0, b]],
                                 rows_v.at[buf], gsems[buf])

        pltpu.sync_copy(zeros_hbm.at[pl.ds(base, rows_per_tile)],
                        agg_sp.at[pl.ds(base, rows_per_tile)])
        plsc.subcore_barrier()

        for g in range(NSLOT):
            load_idx(g, g, isems[g])
        wait_idx(isems[0])
        for b in range(NBUF):
            gather(0, b, b)

        def process_group(ring, nring, regather):
            for b in range(NBUF):
                pltpu.make_async_copy(m0_hbm.at[idx_v.at[ring, 0, b]],
                                      rows_v.at[b], gsems[b]).wait()
                pltpu.sync_copy(rows_v.at[b], agg_sp.at[idx_v.at[ring, 1, b]],
                                add=True)
                if regather:
                    gather(nring, b, b)

        def outer(t, carry):
            for ph in range(NSLOT):
                g = t * NSLOT + ph
                ring, nring = ph, (ph + 1) % NSLOT
                wait_idx(isems[nring])
                process_group(ring, nring, regather=True)
                load_idx(g + NSLOT, ring, isems[ring])
            return carry

        n_full = (n_groups - NSLOT) // NSLOT
        lax.fori_loop(0, n_full, outer, jnp.int32(0))

        g_done = n_full * NSLOT
        n_rem = n_groups - g_done

        for ph in range(2 * NSLOT - 1):
            ring, nring = ph % NSLOT, (ph + 1) % NSLOT

            @pl.when(ph + 1 < n_rem)
            def _(ring=ring, nring=nring):
                wait_idx(isems[nring])
                process_group(ring, nring, regather=True)

            @pl.when(ph + 1 == n_rem)
            def _(ring=ring, nring=nring):
                process_group(ring, nring, regather=False)

            @pl.when(ph + NSLOT < n_rem)
            def _(ph=ph, ring=ring):
                load_idx(g_done + ph + NSLOT, ring, isems[ring])

        plsc.subcore_barrier()
        pltpu.sync_copy(agg_sp.at[pl.ds(base, rows_per_tile)],
                        out_hbm.at[c, pl.ds(base, rows_per_tile)])

    return scatter_kernel


def kernel(x, edge_index, weight, w_ih, w_hh, b_ih, b_hh):
    n, d = x.shape
    e = edge_index.shape[1]

    per_group = NBUF * C
    g_tot = -(-e // per_group)
    e_pad = g_tot * per_group
    g0_total = max(NS * NSLOT, min(g_tot - NS * NSLOT,
                                   round(g_tot * SPLIT0)))
    g1_total = g_tot - g0_total
    q0, r0 = divmod(g0_total, NS)
    q1, r1 = divmod(g1_total, NS)
    assert q0 >= NSLOT and q1 >= NSLOT
    dummy_dst = n
    n_agg = -(-(n + 1) // (NS * 8)) * (NS * 8)
    rows_per_tile = n_agg // NS

    if e_pad > e:
        pad = jnp.zeros((2, e_pad - e), jnp.int32).at[1, :].set(dummy_dst)
        e_arr = jnp.concatenate([edge_index, pad], axis=1)
    else:
        e_arr = edge_index
    e_hbm = e_arr.reshape(2, e_pad // C, C)
    zeros_hbm = jnp.zeros((n_agg, d), jnp.float32)

    br = 2000
    m0, m1 = pl.pallas_call(
        _matmul_body,
        grid=(n // br,),
        in_specs=[pl.BlockSpec((br, d), lambda i: (i, 0)),
                  pl.BlockSpec((d, d), lambda i: (0, 0))],
        out_specs=[pl.BlockSpec((br, d), lambda i: (i, 0)),
                   pl.BlockSpec((br, d), lambda i: (i, 0))],
        out_shape=[jax.ShapeDtypeStruct((n, d), jnp.float32),
                   jax.ShapeDtypeStruct((n, d), jnp.float32)],
    )(x, weight[0])

    partial = _make_scatter_kernel(
        n_agg, d, q0, r0, g0_total, q1, r1, rows_per_tile)(
            m0, m1, e_hbm, zeros_hbm)

    out = pl.pallas_call(
        _gru_body,
        grid=(n // br,),
        in_specs=[
            pl.BlockSpec((br, d), lambda i: (i, 0)),
            pl.BlockSpec((2, br, d), lambda i: (0, i, 0)),
            pl.BlockSpec((d, 3 * d), lambda i: (0, 0)),
            pl.BlockSpec((d, 3 * d), lambda i: (0, 0)),
            pl.BlockSpec((1, 3 * d), lambda i: (0, 0)),
            pl.BlockSpec((1, 3 * d), lambda i: (0, 0)),
        ],
        out_specs=pl.BlockSpec((br, d), lambda i: (i, 0)),
        out_shape=jax.ShapeDtypeStruct((n, d), jnp.float32),
    )(x, partial, w_ih.T, w_hh.T, b_ih.reshape(1, -1), b_hh.reshape(1, -1))

    return out


# speedup vs baseline: 3.5085x; 1.2356x over previous
import functools

import jax
import jax.numpy as jnp
from jax import lax
from jax.experimental import pallas as pl
from jax.experimental.pallas import tpu as pltpu
from jax.experimental.pallas import tpu_sc as plsc

NC = 2
NS = 16
C = 80
NBUF = 4
NSLOT = 3
ZR = 40
SPLIT0 = 0.64


def _matmul_body(x_ref, w_ref, o_ref, o2_ref):
    prod = jnp.dot(x_ref[...], w_ref[...], preferred_element_type=jnp.float32)
    o_ref[...] = prod
    o2_ref[...] = prod


def _gru_body(x_ref, p_ref, wih_ref, whh_ref, bi_ref, bh_ref, o_ref):
    d = x_ref.shape[1]
    xb = x_ref[...]
    agg = p_ref[0] + p_ref[1]
    gi = jnp.dot(agg, wih_ref[...], preferred_element_type=jnp.float32) + bi_ref[...]
    gh = jnp.dot(xb, whh_ref[...], preferred_element_type=jnp.float32) + bh_ref[...]
    i_r, i_z, i_n = gi[:, :d], gi[:, d:2 * d], gi[:, 2 * d:]
    h_r, h_z, h_n = gh[:, :d], gh[:, d:2 * d], gh[:, 2 * d:]
    r = jax.nn.sigmoid(i_r + h_r)
    z = jax.nn.sigmoid(i_z + h_z)
    n = jnp.tanh(i_n + r * h_n)
    h_new = (1.0 - z) * n + z * xb
    o_ref[...] = xb + jnp.maximum(h_new, 0.0)


def _make_scatter_kernel(n_agg, d, q0, r0, g0_total, q1, r1, rows_per_tile):
    mesh = plsc.VectorSubcoreMesh(core_axis_name="c", subcore_axis_name="s",
                                  num_cores=NC, num_subcores=NS)

    @functools.partial(
        pl.kernel,
        out_type=jax.ShapeDtypeStruct((NC, n_agg, d), jnp.float32),
        mesh=mesh,
        scratch_types=[
            pltpu.VMEM_SHARED((n_agg, d), jnp.float32),
            pltpu.VMEM((NSLOT, 2, NBUF, C), jnp.int32),
            pltpu.VMEM((NBUF, C, d), jnp.float32),
            pltpu.VMEM((ZR, d), jnp.float32),
        ] + [pltpu.SemaphoreType.DMA] * (NBUF + NSLOT),
    )
    def scatter_kernel(m0_hbm, m1_hbm, e_hbm, out_hbm,
                       agg_sp, idx_v, rows_v, z_v, *sems):
        e_pad_len = e_hbm.shape[0] // 2
        c = lax.axis_index("c")
        s = lax.axis_index("s")
        gsems, isems = sems[:NBUF], sems[NBUF:]
        base = s * rows_per_tile
        n_groups = lax.select(c == 0, q0 + (s < r0).astype(jnp.int32),
                              q1 + (s < r1).astype(jnp.int32))
        start_g = lax.select(
            c == 0, s * q0 + jnp.minimum(s, r0),
            g0_total + s * q1 + jnp.minimum(s, r1))
        k0 = start_g * NBUF

        def load_idx(g, slot, isem):
            for io in range(2):
                for b in range(NBUF):
                    pltpu.async_copy(
                        e_hbm.at[pl.ds(io * e_pad_len
                                       + (k0 + g * NBUF + b) * C, C)],
                        idx_v.at[slot, io, b], isem)

        def wait_idx(isem):
            for io in range(2):
                for b in range(NBUF):
                    pltpu.make_async_copy(e_hbm.at[pl.ds(0, C)],
                                          idx_v.at[0, io, b], isem).wait()

        def gather(slot, b, buf):
            @pl.when(c == 0)
            def _():
                pltpu.async_copy(m0_hbm.at[idx_v.at[slot, 0, b]],
                                 rows_v.at[buf], gsems[buf])

            @pl.when(c != 0)
            def _():
                pltpu.async_copy(m1_hbm.at[idx_v.at[slot, 0, b]],
                                 rows_v.at[buf], gsems[buf])

        for g in range(NSLOT):
            load_idx(g, g, isems[g])

        def zfill(i, carry):
            for j in range(d // 16):
                z_v[i, pl.ds(j * 16, 16)] = jnp.zeros((16,), jnp.float32)
            return carry

        lax.fori_loop(0, ZR, zfill, jnp.int32(0))
        full, rem = divmod(rows_per_tile, ZR)
        for i in range(full):
            pltpu.sync_copy(z_v, agg_sp.at[pl.ds(base + i * ZR, ZR)])
        if rem:
            pltpu.sync_copy(z_v.at[pl.ds(0, rem)],
                            agg_sp.at[pl.ds(base + full * ZR, rem)])

        wait_idx(isems[0])
        for b in range(NBUF):
            gather(0, b, b)
        plsc.subcore_barrier()

        def process_group(ring, nring, regather):
            for b in range(NBUF):
                pltpu.make_async_copy(m0_hbm.at[idx_v.at[ring, 0, b]],
                                      rows_v.at[b], gsems[b]).wait()
                pltpu.sync_copy(rows_v.at[b], agg_sp.at[idx_v.at[ring, 1, b]],
                                add=True)
                if regather:
                    gather(nring, b, b)

        def outer(t, carry):
            for ph in range(NSLOT):
                g = t * NSLOT + ph
                ring, nring = ph, (ph + 1) % NSLOT
                wait_idx(isems[nring])
                process_group(ring, nring, regather=True)
                load_idx(g + NSLOT, ring, isems[ring])
            return carry

        n_full = (n_groups - NSLOT) // NSLOT
        lax.fori_loop(0, n_full, outer, jnp.int32(0))

        g_done = n_full * NSLOT
        n_rem = n_groups - g_done

        for ph in range(2 * NSLOT - 1):
            ring, nring = ph % NSLOT, (ph + 1) % NSLOT

            @pl.when(ph + 1 < n_rem)
            def _(ring=ring, nring=nring):
                wait_idx(isems[nring])
                process_group(ring, nring, regather=True)

            @pl.when(ph + 1 == n_rem)
            def _(ring=ring, nring=nring):
                process_group(ring, nring, regather=False)

            @pl.when(ph + NSLOT < n_rem)
            def _(ph=ph, ring=ring):
                load_idx(g_done + ph + NSLOT, ring, isems[ring])

        plsc.subcore_barrier()
        pltpu.sync_copy(agg_sp.at[pl.ds(base, rows_per_tile)],
                        out_hbm.at[c, pl.ds(base, rows_per_tile)])

    return scatter_kernel


def kernel(x, edge_index, weight, w_ih, w_hh, b_ih, b_hh):
    n, d = x.shape
    e = edge_index.shape[1]

    per_group = NBUF * C
    g_tot = -(-e // per_group)
    e_pad = g_tot * per_group
    g0_total = max(NS * NSLOT, min(g_tot - NS * NSLOT,
                                   round(g_tot * SPLIT0)))
    g1_total = g_tot - g0_total
    q0, r0 = divmod(g0_total, NS)
    q1, r1 = divmod(g1_total, NS)
    dummy_dst = n
    n_agg = -(-(n + 1) // (NS * 8)) * (NS * 8)
    rows_per_tile = n_agg // NS

    if e_pad > e:
        pad = jnp.zeros((2, e_pad - e), jnp.int32).at[1, :].set(dummy_dst)
        e_hbm = jnp.concatenate([edge_index, pad], axis=1).reshape(-1)
    else:
        e_hbm = edge_index.reshape(-1)

    br = 2000
    m0, m1 = pl.pallas_call(
        _matmul_body,
        grid=(n // br,),
        in_specs=[pl.BlockSpec((br, d), lambda i: (i, 0)),
                  pl.BlockSpec((d, d), lambda i: (0, 0))],
        out_specs=[pl.BlockSpec((br, d), lambda i: (i, 0)),
                   pl.BlockSpec((br, d), lambda i: (i, 0))],
        out_shape=[jax.ShapeDtypeStruct((n, d), jnp.float32),
                   jax.ShapeDtypeStruct((n, d), jnp.float32)],
    )(x, weight[0])

    partial = _make_scatter_kernel(
        n_agg, d, q0, r0, g0_total, q1, r1, rows_per_tile)(
            m0, m1, e_hbm)

    out = pl.pallas_call(
        _gru_body,
        grid=(n // br,),
        in_specs=[
            pl.BlockSpec((br, d), lambda i: (i, 0)),
            pl.BlockSpec((2, br, d), lambda i: (0, i, 0)),
            pl.BlockSpec((d, 3 * d), lambda i: (0, 0)),
            pl.BlockSpec((d, 3 * d), lambda i: (0, 0)),
            pl.BlockSpec((1, 3 * d), lambda i: (0, 0)),
            pl.BlockSpec((1, 3 * d), lambda i: (0, 0)),
        ],
        out_specs=pl.BlockSpec((br, d), lambda i: (i, 0)),
        out_shape=jax.ShapeDtypeStruct((n, d), jnp.float32),
    )(x, partial, w_ih.T, w_hh.T, b_ih.reshape(1, -1), b_hh.reshape(1, -1))

    return out


# speedup vs baseline: 4.0240x; 1.1469x over previous
import functools

import jax
import jax.numpy as jnp
from jax import lax
from jax.experimental import pallas as pl
from jax.experimental.pallas import tpu as pltpu
from jax.experimental.pallas import tpu_sc as plsc

NC = 2
NS = 16
C = 80
NBUF = 4
NSLOT = 3
ZR = 40
SPLIT0 = 0.50


def _matmul_body(x_ref, w_ref, o_ref, o2_ref):
    prod = jnp.dot(x_ref[...], w_ref[...], preferred_element_type=jnp.float32)
    o_ref[...] = prod
    o2_ref[...] = prod


def _gru_body(x_ref, p0_ref, p1_ref, wih_ref, whh_ref, bi_ref, bh_ref, o_ref):
    d = x_ref.shape[1]
    xb = x_ref[...]
    agg = p0_ref[...] + p1_ref[...]
    gi = jnp.dot(agg, wih_ref[...], preferred_element_type=jnp.float32) + bi_ref[...]
    gh = jnp.dot(xb, whh_ref[...], preferred_element_type=jnp.float32) + bh_ref[...]
    i_r, i_z, i_n = gi[:, :d], gi[:, d:2 * d], gi[:, 2 * d:]
    h_r, h_z, h_n = gh[:, :d], gh[:, d:2 * d], gh[:, 2 * d:]
    r = jax.nn.sigmoid(i_r + h_r)
    z = jax.nn.sigmoid(i_z + h_z)
    n = jnp.tanh(i_n + r * h_n)
    h_new = (1.0 - z) * n + z * xb
    o_ref[...] = xb + jnp.maximum(h_new, 0.0)


def _make_scatter_kernel(n_agg, d, q0, r0, g0_total, q1, r1, rows_per_tile):
    mesh = plsc.VectorSubcoreMesh(core_axis_name="c", subcore_axis_name="s",
                                  num_cores=NC, num_subcores=NS)

    @functools.partial(
        pl.kernel,
        out_type=[jax.ShapeDtypeStruct((n_agg, d), jnp.float32),
                  jax.ShapeDtypeStruct((n_agg, d), jnp.float32)],
        mesh=mesh,
        scratch_types=[
            pltpu.VMEM_SHARED((n_agg, d), jnp.float32),
            pltpu.VMEM((NSLOT, 2, NBUF, C), jnp.int32),
            pltpu.VMEM((NBUF, C, d), jnp.float32),
            pltpu.VMEM((ZR, d), jnp.float32),
        ] + [pltpu.SemaphoreType.DMA] * (NBUF + NSLOT),
    )
    def scatter_kernel(m0_hbm, m1_hbm, e_hbm, out0_hbm, out1_hbm,
                       agg_sp, idx_v, rows_v, z_v, *sems):
        e_pad_len = e_hbm.shape[0] // 2
        c = lax.axis_index("c")
        s = lax.axis_index("s")
        gsems, isems = sems[:NBUF], sems[NBUF:]
        base = s * rows_per_tile
        n_groups = lax.select(c == 0, q0 + (s < r0).astype(jnp.int32),
                              q1 + (s < r1).astype(jnp.int32))
        start_g = lax.select(
            c == 0, s * q0 + jnp.minimum(s, r0),
            g0_total + s * q1 + jnp.minimum(s, r1))
        k0 = start_g * NBUF

        def load_idx(g, slot, isem):
            for io in range(2):
                for b in range(NBUF):
                    pltpu.async_copy(
                        e_hbm.at[pl.ds(io * e_pad_len
                                       + (k0 + g * NBUF + b) * C, C)],
                        idx_v.at[slot, io, b], isem)

        def wait_idx(isem):
            for io in range(2):
                for b in range(NBUF):
                    pltpu.make_async_copy(e_hbm.at[pl.ds(0, C)],
                                          idx_v.at[0, io, b], isem).wait()

        def gather(slot, b, buf):
            @pl.when(c == 0)
            def _():
                pltpu.async_copy(m0_hbm.at[idx_v.at[slot, 0, b]],
                                 rows_v.at[buf], gsems[buf])

            @pl.when(c != 0)
            def _():
                pltpu.async_copy(m1_hbm.at[idx_v.at[slot, 0, b]],
                                 rows_v.at[buf], gsems[buf])

        for g in range(NSLOT):
            load_idx(g, g, isems[g])

        def zfill(i, carry):
            for j in range(d // 16):
                z_v[i, pl.ds(j * 16, 16)] = jnp.zeros((16,), jnp.float32)
            return carry

        lax.fori_loop(0, ZR, zfill, jnp.int32(0))
        full, rem = divmod(rows_per_tile, ZR)
        for i in range(full):
            pltpu.sync_copy(z_v, agg_sp.at[pl.ds(base + i * ZR, ZR)])
        if rem:
            pltpu.sync_copy(z_v.at[pl.ds(0, rem)],
                            agg_sp.at[pl.ds(base + full * ZR, rem)])

        wait_idx(isems[0])
        for b in range(NBUF):
            gather(0, b, b)
        plsc.subcore_barrier()

        def process_group(ring, nring, regather):
            for b in range(NBUF):
                pltpu.make_async_copy(m0_hbm.at[idx_v.at[ring, 0, b]],
                                      rows_v.at[b], gsems[b]).wait()
                pltpu.sync_copy(rows_v.at[b], agg_sp.at[idx_v.at[ring, 1, b]],
                                add=True)
                if regather:
                    gather(nring, b, b)

        def outer(t, carry):
            for ph in range(NSLOT):
                g = t * NSLOT + ph
                ring, nring = ph, (ph + 1) % NSLOT
                wait_idx(isems[nring])
                process_group(ring, nring, regather=True)
                load_idx(g + NSLOT, ring, isems[ring])
            return carry

        n_full = (n_groups - NSLOT) // NSLOT
        lax.fori_loop(0, n_full, outer, jnp.int32(0))

        g_done = n_full * NSLOT
        n_rem = n_groups - g_done

        for ph in range(2 * NSLOT - 1):
            ring, nring = ph % NSLOT, (ph + 1) % NSLOT

            @pl.when(ph + 1 < n_rem)
            def _(ring=ring, nring=nring):
                wait_idx(isems[nring])
                process_group(ring, nring, regather=True)

            @pl.when(ph + 1 == n_rem)
            def _(ring=ring, nring=nring):
                process_group(ring, nring, regather=False)

            @pl.when(ph + NSLOT < n_rem)
            def _(ph=ph, ring=ring):
                load_idx(g_done + ph + NSLOT, ring, isems[ring])

        plsc.subcore_barrier()

        @pl.when(c == 0)
        def _():
            pltpu.sync_copy(agg_sp.at[pl.ds(base, rows_per_tile)],
                            out0_hbm.at[pl.ds(base, rows_per_tile)])

        @pl.when(c != 0)
        def _():
            pltpu.sync_copy(agg_sp.at[pl.ds(base, rows_per_tile)],
                            out1_hbm.at[pl.ds(base, rows_per_tile)])

    return scatter_kernel


def kernel(x, edge_index, weight, w_ih, w_hh, b_ih, b_hh):
    n, d = x.shape
    e = edge_index.shape[1]

    per_group = NBUF * C
    g_tot = -(-e // per_group)
    e_pad = g_tot * per_group
    g0_total = max(NS * NSLOT, min(g_tot - NS * NSLOT,
                                   round(g_tot * SPLIT0)))
    g1_total = g_tot - g0_total
    q0, r0 = divmod(g0_total, NS)
    q1, r1 = divmod(g1_total, NS)
    dummy_dst = n
    n_agg = -(-(n + 1) // (NS * 8)) * (NS * 8)
    rows_per_tile = n_agg // NS

    if e_pad > e:
        pad = jnp.zeros((2, e_pad - e), jnp.int32).at[1, :].set(dummy_dst)
        e_hbm = jnp.concatenate([edge_index, pad], axis=1).reshape(-1)
    else:
        e_hbm = edge_index.reshape(-1)

    br = 2000
    m0, m1 = pl.pallas_call(
        _matmul_body,
        grid=(n // br,),
        in_specs=[pl.BlockSpec((br, d), lambda i: (i, 0)),
                  pl.BlockSpec((d, d), lambda i: (0, 0))],
        out_specs=[pl.BlockSpec((br, d), lambda i: (i, 0)),
                   pl.BlockSpec((br, d), lambda i: (i, 0))],
        out_shape=[jax.ShapeDtypeStruct((n, d), jnp.float32),
                   jax.ShapeDtypeStruct((n, d), jnp.float32)],
    )(x, weight[0])

    p0, p1 = _make_scatter_kernel(
        n_agg, d, q0, r0, g0_total, q1, r1, rows_per_tile)(
            m0, m1, e_hbm)

    out = pl.pallas_call(
        _gru_body,
        grid=(n // br,),
        in_specs=[
            pl.BlockSpec((br, d), lambda i: (i, 0)),
            pl.BlockSpec((br, d), lambda i: (i, 0)),
            pl.BlockSpec((br, d), lambda i: (i, 0)),
            pl.BlockSpec((d, 3 * d), lambda i: (0, 0)),
            pl.BlockSpec((d, 3 * d), lambda i: (0, 0)),
            pl.BlockSpec((1, 3 * d), lambda i: (0, 0)),
            pl.BlockSpec((1, 3 * d), lambda i: (0, 0)),
        ],
        out_specs=pl.BlockSpec((br, d), lambda i: (i, 0)),
        out_shape=jax.ShapeDtypeStruct((n, d), jnp.float32),
    )(x, p0, p1, w_ih.T, w_hh.T, b_ih.reshape(1, -1), b_hh.reshape(1, -1))

    return out
